# baseline (speedup 1.0000x reference)
import functools

import jax
import jax.numpy as jnp
from jax import lax
from jax.experimental import pallas as pl
from jax.experimental.pallas import tpu as pltpu
from jax.experimental.pallas import tpu_sc as plsc

_LANES = 16
_NUM_WORKERS = 32
_COLS_PER_WORKER = 512
_SC_ROWS = 40
_SC_CHUNK_ROWS = 40
_TC_BLOCK_ROWS = 40
_TC_BLOCK_COLS = 2048


def _sc_run(kT, loc, scale, eT, n_cols):
    n_chunks = _SC_ROWS // _SC_CHUNK_ROWS
    mesh = plsc.VectorSubcoreMesh(core_axis_name="c", subcore_axis_name="s")

    @functools.partial(
        pl.kernel,
        mesh=mesh,
        compiler_params=pltpu.CompilerParams(
            needs_layout_passes=False, use_tc_tiling_on_sc=True),
        out_type=jax.ShapeDtypeStruct((_SC_ROWS, n_cols), jnp.float32),
        scratch_types=[
            pltpu.VMEM((8,), jnp.float32),
            pltpu.VMEM((8,), jnp.float32),
            pltpu.VMEM((_SC_CHUNK_ROWS, _COLS_PER_WORKER), jnp.int32),
            pltpu.VMEM((_SC_CHUNK_ROWS, _COLS_PER_WORKER), jnp.int32),
            pltpu.VMEM((_SC_CHUNK_ROWS, _COLS_PER_WORKER), jnp.float32),
            pltpu.VMEM((_SC_CHUNK_ROWS, _COLS_PER_WORKER), jnp.float32),
            pltpu.VMEM((_SC_CHUNK_ROWS, _COLS_PER_WORKER), jnp.float32),
            pltpu.VMEM((_SC_CHUNK_ROWS, _COLS_PER_WORKER), jnp.float32),
            pltpu.SemaphoreType.DMA,
            pltpu.SemaphoreType.DMA,
            pltpu.SemaphoreType.DMA,
            pltpu.SemaphoreType.DMA,
        ],
    )
    def run(k_hbm, loc_hbm, scale_hbm, eps_hbm, out_hbm,
            tloc, tscl, kb0, kb1, eb0, eb1, ob0, ob1,
            isem0, isem1, osem0, osem1):
        wid = lax.axis_index("s") * 2 + lax.axis_index("c")
        cols = pl.ds(wid * _COLS_PER_WORKER, _COLS_PER_WORKER)
        pltpu.sync_copy(loc_hbm, tloc)
        pltpu.sync_copy(scale_hbm, tscl)

        kb, eb, ob = (kb0, kb1), (eb0, eb1), (ob0, ob1)
        isems = (isem0, isem1)
        osems = (osem0, osem1)

        def start_in(g):
            slot = g % 2
            rows = pl.ds(g * _SC_CHUNK_ROWS, _SC_CHUNK_ROWS)
            sem = isems[slot]
            return (
                pltpu.async_copy(k_hbm.at[rows, cols], kb[slot], sem),
                pltpu.async_copy(eps_hbm.at[rows, cols], eb[slot], sem),
            )

        pending_in = {0: start_in(0)}
        pending_out = {}
        for g in range(n_chunks):
            slot = g % 2
            if g + 1 < n_chunks:
                pending_in[g + 1] = start_in(g + 1)
            for h in pending_in.pop(g):
                h.wait()
            if g >= 2:
                pending_out.pop(g - 2).wait()

            kbs, ebs, obs = kb[slot], eb[slot], ob[slot]

            @plsc.parallel_loop(0, _SC_CHUNK_ROWS, 1)
            def _(r):
                for cc in range(_COLS_PER_WORKER // _LANES):
                    s = pl.ds(cc * _LANES, _LANES)
                    kv = kbs[r, s]
                    obs[r, s] = (plsc.load_gather(tloc, [kv])
                                 + plsc.load_gather(tscl, [kv]) * ebs[r, s])

            rows = pl.ds(g * _SC_CHUNK_ROWS, _SC_CHUNK_ROWS)
            pending_out[g] = pltpu.async_copy(
                obs, out_hbm.at[rows, cols], osems[slot])
        for h in pending_out.values():
            h.wait()

    return run(kT, loc, scale, eT)


def _tc_run(kT, loc, scale, eT, n_rows, n_cols):
    tc_rows = n_rows - _SC_ROWS
    row0 = _SC_ROWS // _TC_BLOCK_ROWS
    grid = (tc_rows // _TC_BLOCK_ROWS, n_cols // _TC_BLOCK_COLS)

    def body(loc_ref, scl_ref, k_ref, e_ref, o_ref):
        kv = k_ref[...]
        ev = e_ref[...]
        acc = jnp.zeros_like(ev)
        for i in range(8):
            acc = jnp.where(kv == i, loc_ref[i] + scl_ref[i] * ev, acc)
        o_ref[...] = acc

    spec = pl.BlockSpec((_TC_BLOCK_ROWS, _TC_BLOCK_COLS),
                        lambda i, j: (row0 + i, j))
    return pl.pallas_call(
        body,
        grid=grid,
        in_specs=[
            pl.BlockSpec(memory_space=pltpu.SMEM),
            pl.BlockSpec(memory_space=pltpu.SMEM),
            spec,
            spec,
        ],
        out_specs=spec,
        out_shape=jax.ShapeDtypeStruct((n_rows, n_cols), jnp.float32),
    )(loc, scale, kT, eT)


def kernel(k, loc, scale, eps):
    n_rows, n_cols = k.shape
    kT = k.astype(jnp.int32).T
    eT = eps.astype(jnp.float32).T
    loc = loc.astype(jnp.float32)
    scale = scale.astype(jnp.float32)
    sc_part = _sc_run(kT, loc, scale, eT, n_rows)
    tc_full = _tc_run(kT, loc, scale, eT, n_cols, n_rows)
    outT = lax.dynamic_update_slice(tc_full, sc_part, (0, 0))
    return outT.T


# speedup vs baseline: 2.2657x; 2.2657x over previous
import functools

import jax
import jax.numpy as jnp
from jax.experimental import pallas as pl
from jax.experimental.pallas import tpu as pltpu

_BLOCK_COLS = 4096


def _tc_run(kT, loc16, scale16, eT, n_rows, n_cols):
    grid = (n_cols // _BLOCK_COLS,)

    def body(loc_ref, scl_ref, k_ref, e_ref, o_ref):
        kv = k_ref[...]
        ev = e_ref[...]
        acc = jnp.zeros_like(ev)
        for i in range(8):
            acc = jnp.where(kv == i, loc_ref[i] + scl_ref[i] * ev, acc)
        o_ref[...] = acc

    return pl.pallas_call(
        body,
        grid=grid,
        in_specs=[
            pl.BlockSpec(memory_space=pltpu.SMEM),
            pl.BlockSpec(memory_space=pltpu.SMEM),
            pl.BlockSpec((n_rows, _BLOCK_COLS), lambda i: (0, i)),
            pl.BlockSpec((n_rows, _BLOCK_COLS), lambda i: (0, i)),
        ],
        out_specs=pl.BlockSpec((n_rows, _BLOCK_COLS), lambda i: (0, i)),
        out_shape=jax.ShapeDtypeStruct((n_rows, n_cols), jnp.float32),
    )(loc16, scale16, kT, eT)


def kernel(k, loc, scale, eps):
    n_rows, n_cols = k.shape
    loc16 = jnp.zeros((16,), jnp.float32).at[: loc.shape[0]].set(loc)
    scale16 = jnp.zeros((16,), jnp.float32).at[: scale.shape[0]].set(scale)
    outT = _tc_run(k.astype(jnp.int32).T, loc16, scale16,
                   eps.astype(jnp.float32).T, n_cols, n_rows)
    return outT.T
